# baseline (speedup 1.0000x reference)
import jax
import jax.numpy as jnp
from jax.experimental import pallas as pl
from jax.experimental.pallas import tpu as pltpu

_NE = 1024
_D = 64
_HW = 1024
_B = 32
_NB = 4
_NELEM = _B * _D * _HW


def _vq_body(x_ref, w_ref, q_ref, loss_ref, acc_ref):
    i = pl.program_id(0)
    n = pl.num_programs(0)
    w = w_ref[...]
    w2 = w + w
    wb = w.astype(jnp.bfloat16)
    wsq = jnp.sum(w * w, axis=1, keepdims=True)
    part = jnp.zeros((1, 1), jnp.float32)
    for j in range(_NB):
        xc = x_ref[j].reshape(_D, _HW)
        xsq = jnp.sum(xc * xc, axis=0, keepdims=True)
        m2 = jax.lax.dot_general(w2, xc, (((1,), (0,)), ((), ())),
                                 preferred_element_type=jnp.float32)
        dist = (xsq + wsq) - m2
        dmin = jnp.min(dist, axis=0, keepdims=True)
        ids = jax.lax.broadcasted_iota(jnp.int32, dist.shape, 0)
        idx = jnp.min(jnp.where(dist == dmin, ids, _NE), axis=0, keepdims=True)
        oh = (ids == idx).astype(jnp.bfloat16)
        qc = jax.lax.dot_general(wb, oh, (((0,), (0,)), ((), ())),
                                 preferred_element_type=jnp.float32)
        q_ref[j] = (xc + (qc - xc)).reshape(_D, 32, 32)
        diff = qc - xc
        part = part + jnp.sum(jnp.sum(diff * diff, axis=1, keepdims=True),
                              axis=0, keepdims=True)

    @pl.when(i == 0)
    def _init():
        acc_ref[...] = part

    @pl.when(i > 0)
    def _acc():
        acc_ref[...] += part

    @pl.when(i == n - 1)
    def _fini():
        mean = acc_ref[0, 0] * (1.0 / _NELEM)
        loss_ref[0, 0] = mean + 0.25 * mean


def kernel(inputs, W):
    q, loss = pl.pallas_call(
        _vq_body,
        grid=(_B // _NB,),
        in_specs=[
            pl.BlockSpec((_NB, _D, 32, 32), lambda i: (i, 0, 0, 0)),
            pl.BlockSpec((_NE, _D), lambda i: (0, 0)),
        ],
        out_specs=[
            pl.BlockSpec((_NB, _D, 32, 32), lambda i: (i, 0, 0, 0)),
            pl.BlockSpec(memory_space=pltpu.SMEM),
        ],
        out_shape=[
            jax.ShapeDtypeStruct((_B, _D, 32, 32), jnp.float32),
            jax.ShapeDtypeStruct((1, 1), jnp.float32),
        ],
        scratch_shapes=[pltpu.VMEM((1, 1), jnp.float32)],
    )(inputs, W)
    return q, loss[0, 0]


# speedup vs baseline: 1.1835x; 1.1835x over previous
import functools

import jax
import jax.numpy as jnp
from jax import lax
from jax.experimental import pallas as pl
from jax.experimental.pallas import tpu as pltpu
from jax.experimental.pallas import tpu_sc as plsc

_NE = 1024
_D = 64
_HW = 1024
_B = 32
_NB = 8
_NELEM = _B * _D * _HW
_TCHUNK = 256


def _vq_body(x_ref, w_ref, idx_ref, loss_ref, acc_ref):
    i = pl.program_id(0)
    n = pl.num_programs(0)
    w = w_ref[...]
    w2 = w + w
    wsq = jnp.sum(w * w, axis=1, keepdims=True)
    part = jnp.zeros((1, 1), jnp.float32)
    for j in range(_NB):
        xc = x_ref[j]
        xsq = jnp.sum(xc * xc, axis=0, keepdims=True)
        m2 = jax.lax.dot_general(w2, xc, (((1,), (0,)), ((), ())),
                                 preferred_element_type=jnp.float32)
        dist = (xsq + wsq) - m2
        dmin = jnp.min(dist, axis=0, keepdims=True)
        ids = jax.lax.broadcasted_iota(jnp.int32, dist.shape, 0)
        idx = jnp.min(jnp.where(dist == dmin, ids, _NE), axis=0, keepdims=True)
        idx_ref[pl.ds(j, 1), :] = idx
        part = part + jnp.sum(dmin, axis=1, keepdims=True)

    @pl.when(i == 0)
    def _init():
        acc_ref[...] = part

    @pl.when(i > 0)
    def _acc():
        acc_ref[...] += part

    @pl.when(i == n - 1)
    def _fini():
        mean = acc_ref[0, 0] * (1.0 / _NELEM)
        loss_ref[0, 0] = mean + 0.25 * mean


def _sc_gather(w_hbm, idx_hbm, out_hbm, idx_a, idx_b, rows_v, sem):
    c = lax.axis_index("c")
    s = lax.axis_index("s")
    wid = s * 2 + c
    base = wid * _HW
    half = _HW // 2
    pltpu.sync_copy(idx_hbm.at[pl.ds(base, half)], idx_a)
    pltpu.sync_copy(idx_hbm.at[pl.ds(base + half, half)], idx_b)
    pltpu.async_copy(w_hbm.at[idx_a], rows_v, sem).wait()
    pltpu.sync_copy(rows_v, out_hbm.at[pl.ds(base, half)])
    pltpu.async_copy(w_hbm.at[idx_b], rows_v, sem).wait()
    pltpu.sync_copy(rows_v, out_hbm.at[pl.ds(base + half, half)])


def kernel(inputs, W):
    shp = inputs.shape
    x3 = inputs.reshape(_B, _D, _HW)
    idx, loss = pl.pallas_call(
        _vq_body,
        grid=(_B // _NB,),
        in_specs=[
            pl.BlockSpec((_NB, _D, _HW), lambda i: (i, 0, 0)),
            pl.BlockSpec((_NE, _D), lambda i: (0, 0)),
        ],
        out_specs=[
            pl.BlockSpec((_NB, _HW), lambda i: (i, 0)),
            pl.BlockSpec(memory_space=pltpu.SMEM),
        ],
        out_shape=[
            jax.ShapeDtypeStruct((_B, _HW), jnp.int32),
            jax.ShapeDtypeStruct((1, 1), jnp.float32),
        ],
        scratch_shapes=[pltpu.VMEM((1, 1), jnp.float32)],
    )(x3, W)

    wp = jnp.pad(W, ((0, 0), (0, 128 - _D)))
    mesh = plsc.VectorSubcoreMesh(core_axis_name="c", subcore_axis_name="s")
    gat = functools.partial(
        pl.kernel, mesh=mesh,
        out_type=jax.ShapeDtypeStruct((_B * _HW, 128), jnp.float32),
        scratch_types=[
            pltpu.VMEM((_HW // 2,), jnp.int32),
            pltpu.VMEM((_HW // 2,), jnp.int32),
            pltpu.VMEM((_HW // 2, 128), jnp.float32),
            pltpu.SemaphoreType.DMA,
        ],
    )(_sc_gather)
    qf = gat(wp, idx.reshape(-1))
    q = jnp.transpose(qf.reshape(_B, 32, 32, 128)[..., :_D], (0, 3, 1, 2))
    return q, loss[0, 0]


# speedup vs baseline: 1.4922x; 1.2609x over previous
import jax
import jax.numpy as jnp
from jax.experimental import pallas as pl
from jax.experimental.pallas import tpu as pltpu

_NE = 1024
_D = 64
_HW = 1024
_B = 32
_NB = 8
_NELEM = _B * _D * _HW


def _vq_body(x_ref, w_ref, q_ref, loss_ref, acc_ref):
    i = pl.program_id(0)
    n = pl.num_programs(0)
    w = w_ref[...]
    w2 = w + w
    wb = w.astype(jnp.bfloat16)
    wsq = jnp.sum(w * w, axis=1, keepdims=True)
    part = jnp.zeros((1, 1), jnp.float32)
    for j in range(_NB):
        xc = x_ref[j]
        xsq = jnp.sum(xc * xc, axis=0, keepdims=True)
        m2 = jax.lax.dot_general(w2, xc, (((1,), (0,)), ((), ())),
                                 preferred_element_type=jnp.float32)
        dist = (xsq + wsq) - m2
        dmin = jnp.min(dist, axis=0, keepdims=True)
        ids = jax.lax.broadcasted_iota(jnp.int32, dist.shape, 0)
        idx = jnp.min(jnp.where(dist == dmin, ids, _NE), axis=0, keepdims=True)
        oh = (ids == idx).astype(jnp.bfloat16)
        qc = jax.lax.dot_general(wb, oh, (((0,), (0,)), ((), ())),
                                 preferred_element_type=jnp.float32)
        q_ref[j] = xc + (qc - xc)
        diff = qc - xc
        part = part + jnp.sum(jnp.sum(diff * diff, axis=1, keepdims=True),
                              axis=0, keepdims=True)

    @pl.when(i == 0)
    def _init():
        acc_ref[...] = part

    @pl.when(i > 0)
    def _acc():
        acc_ref[...] += part

    @pl.when(i == n - 1)
    def _fini():
        mean = acc_ref[0, 0] * (1.0 / _NELEM)
        loss_ref[0, 0] = mean + 0.25 * mean


def kernel(inputs, W):
    shp = inputs.shape
    x3 = inputs.reshape(_B, _D, _HW)
    q, loss = pl.pallas_call(
        _vq_body,
        grid=(_B // _NB,),
        in_specs=[
            pl.BlockSpec((_NB, _D, _HW), lambda i: (i, 0, 0)),
            pl.BlockSpec((_NE, _D), lambda i: (0, 0)),
        ],
        out_specs=[
            pl.BlockSpec((_NB, _D, _HW), lambda i: (i, 0, 0)),
            pl.BlockSpec(memory_space=pltpu.SMEM),
        ],
        out_shape=[
            jax.ShapeDtypeStruct((_B, _D, _HW), jnp.float32),
            jax.ShapeDtypeStruct((1, 1), jnp.float32),
        ],
        scratch_shapes=[pltpu.VMEM((1, 1), jnp.float32)],
    )(x3, W)
    return q.reshape(shp), loss[0, 0]
